# baseline (speedup 1.0000x reference)
import jax
import jax.numpy as jnp
from jax import lax
from jax.experimental import pallas as pl
from jax.experimental.pallas import tpu as pltpu
from jax.experimental.pallas import tpu_sc as plsc

_NF = 26
_VOC = 100001
_DIM = 16
_B = 16384
_NC = 2
_NS = 16
_NW = _NC * _NS
_BPW = _B // _NW
_LANES = 16
_GCH = 128
_HB = _BPW // 2

_RC = 1280
_LPB = _RC // 8
_NBLK = _VOC // _RC
_NLINE = _NBLK * _LPB
_TAIL0 = _NLINE * 8
_TROWS = 32
_ITS = 3


def _repack_body(tab_hbm, pk_hbm, s0, s1, o0, o1, sem_s0, sem_s1,
                 sem_w0, sem_w1):
    wid = lax.axis_index("s") * _NC + lax.axis_index("c")
    lanes = lax.iota(jnp.int32, _LANES)

    sbufs = (s0, s1)
    obufs = (o0, o1)
    sem_s = (sem_s0, sem_s1)
    sem_w = (sem_w0, sem_w1)

    def _params(it):
        bi = wid + it * _NW
        valid = bi < _NBLK
        rc = pl.multiple_of(jnp.where(valid, bi, 0) * _RC, 128)
        gd = pl.multiple_of(jnp.where(valid, bi * _LPB, _NLINE), 8)
        return rc, gd

    def _fbody(f, carry):
        def _fire_stage(it):
            rc, _ = _params(it)
            return pltpu.async_copy(
                tab_hbm.at[f, :, pl.ds(rc, _RC)],
                sbufs[it % 2].at[:, pl.ds(0, _RC)],
                sem_s[it % 2]
            )

        def _extract(sb, ob):
            def _line(gl, carry2, sb=sb, ob=ob):
                for j in range(8):
                    col = gl * 8 + j
                    vals = plsc.load_gather(
                        sb, [lanes, jnp.full((_LANES,), 0, jnp.int32) + col]
                    )
                    ob[gl, pl.ds(j * _LANES, _LANES)] = vals
                return carry2

            lax.fori_loop(0, _LPB, _line, 0)

        def _fire_write(it):
            _, gd = _params(it)
            return pltpu.async_copy(
                obufs[it % 2], pk_hbm.at[f, pl.ds(gd, _LPB), :],
                sem_w[it % 2]
            )

        cp0 = _fire_stage(0)
        cp1 = _fire_stage(1)
        cp0.wait()
        _extract(sbufs[0], obufs[0])
        w0 = _fire_write(0)
        cp2 = _fire_stage(2)
        cp1.wait()
        _extract(sbufs[1], obufs[1])
        w1 = _fire_write(1)
        cp2.wait()
        w0.wait()
        _extract(sbufs[0], obufs[0])
        w2 = _fire_write(2)
        w1.wait()
        w2.wait()
        return carry

    lax.fori_loop(0, _NF, _fbody, 0)


def _lookup_body(idx_hbm, pk_hbm, tail_hbm, out_hbm,
                 idx_v, g_v, gs_v, s_v, rows_v, acc_v, sem):
    wid = lax.axis_index("s") * _NC + lax.axis_index("c")
    base = wid * _BPW

    seven = jnp.full((_LANES,), 7, jnp.int32)
    gmax = jnp.full((_LANES,), _NLINE - 1, jnp.int32)
    nchunk = _BPW // _LANES
    hchunk = _HB // _LANES
    inv = jnp.full((_LANES,), 1.0 / _NF, jnp.float32)
    lanes = lax.iota(jnp.int32, _LANES)

    zero = jnp.full((_LANES,), 0.0, jnp.float32)

    def _zero(j, carry):
        for d in range(_DIM):
            acc_v[d, pl.ds(j * _LANES, _LANES)] = zero
        return carry

    lax.fori_loop(0, nchunk, _zero, 0)

    def _feature(f, carry):
        pltpu.sync_copy(idx_hbm.at[f, pl.ds(base, _BPW)], idx_v)
        pltpu.sync_copy(tail_hbm.at[f], rows_v.at[pl.ds(_HB, _TROWS)])

        def _split(j, carry2):
            s = pl.ds(j * _LANES, _LANES)
            v = idx_v[s]
            g = lax.shift_right_logical(v, 3)
            g_v[s] = g
            gs_v[s] = jnp.minimum(g, gmax)
            s_v[s] = lax.shift_left(lax.bitwise_and(v, seven), 4)
            return carry2

        lax.fori_loop(0, nchunk, _split, 0)

        for h in range(2):
            copies = [
                pltpu.async_copy(
                    pk_hbm.at[f].at[gs_v.at[pl.ds(h * _HB + c * _GCH, _GCH)]],
                    rows_v.at[pl.ds(c * _GCH, _GCH)],
                    sem,
                )
                for c in range(_HB // _GCH)
            ]
            for cp in copies:
                cp.wait()

            def _ext(j, carry2, h=h):
                sl = pl.ds(h * _HB + j * _LANES, _LANES)
                g16 = g_v[sl]
                rows16 = jnp.where(
                    g16 > gmax, g16 + (_HB - _NLINE), lanes + j * _LANES
                )
                cols = s_v[sl]
                for d in range(_DIM):
                    acc_v[d, sl] = acc_v[d, sl] + plsc.load_gather(
                        rows_v, [rows16, cols + d]
                    )
                return carry2

            lax.fori_loop(0, hchunk, _ext, 0)
        return carry

    lax.fori_loop(0, _NF, _feature, 0)

    def _scale(j, carry):
        for d in range(_DIM):
            sl = pl.ds(j * _LANES, _LANES)
            acc_v[d, sl] = acc_v[d, sl] * inv
        return carry

    lax.fori_loop(0, nchunk, _scale, 0)
    pltpu.sync_copy(acc_v, out_hbm.at[:, pl.ds(base, _BPW)])


def kernel(inputs, batch_size, tables):
    del batch_size
    idx_t = inputs.T
    tab_t = jnp.transpose(tables, (0, 2, 1))
    tab_tail = jnp.pad(
        lax.slice(tables, (0, _TAIL0, 0), (_NF, _VOC, _DIM)),
        ((0, 0), (0, _TROWS * 8 - (_VOC - _TAIL0)), (0, 0)),
    ).reshape(_NF, _TROWS, 128)

    repack = pl.kernel(
        _repack_body,
        out_type=jax.ShapeDtypeStruct((_NF, _NLINE + _LPB, 128), jnp.float32),
        mesh=plsc.VectorSubcoreMesh(core_axis_name="c", subcore_axis_name="s"),
        scratch_types=[
            pltpu.VMEM((_DIM, _RC + 1), jnp.float32),
            pltpu.VMEM((_DIM, _RC + 1), jnp.float32),
            pltpu.VMEM((_LPB, 128), jnp.float32),
            pltpu.VMEM((_LPB, 128), jnp.float32),
            pltpu.SemaphoreType.DMA,
            pltpu.SemaphoreType.DMA,
            pltpu.SemaphoreType.DMA,
            pltpu.SemaphoreType.DMA,
        ],
        compiler_params=pltpu.CompilerParams(
            use_tc_tiling_on_sc=True, needs_layout_passes=False
        ),
    )
    packed = repack(tab_t)

    lookup = pl.kernel(
        _lookup_body,
        out_type=jax.ShapeDtypeStruct((_DIM, _B), jnp.float32),
        mesh=plsc.VectorSubcoreMesh(core_axis_name="c", subcore_axis_name="s"),
        scratch_types=[
            pltpu.VMEM((_BPW,), jnp.int32),
            pltpu.VMEM((_BPW,), jnp.int32),
            pltpu.VMEM((_BPW,), jnp.int32),
            pltpu.VMEM((_BPW,), jnp.int32),
            pltpu.VMEM((_HB + _TROWS, 128), jnp.float32),
            pltpu.VMEM((_DIM, _BPW), jnp.float32),
            pltpu.SemaphoreType.DMA,
        ],
        compiler_params=pltpu.CompilerParams(
            use_tc_tiling_on_sc=True, needs_layout_passes=False
        ),
    )
    out_t = lookup(idx_t, packed, tab_tail)
    return out_t.T


# speedup vs baseline: 2.1792x; 2.1792x over previous
import jax
import jax.numpy as jnp
from jax import lax
from jax.experimental import pallas as pl
from jax.experimental.pallas import tpu as pltpu
from jax.experimental.pallas import tpu_sc as plsc

_NF = 26
_VOC = 100001
_DIM = 16
_B = 16384
_NC = 2
_NS = 16
_NW = _NC * _NS
_BPW = _B // _NW
_LANES = 16
_GCH = 128
_HB = _BPW // 2

_RC = 1280
_LPB = _RC // 8
_NBLK = _VOC // _RC
_NLINE = _NBLK * _LPB
_TAIL0 = _NLINE * 8
_TROWS = 32
_ITS = 3


def _repack_body(tab_hbm, pk_hbm, s0, s1, o0, o1, sem_s0, sem_s1,
                 sem_w0, sem_w1):
    wid = lax.axis_index("s") * _NC + lax.axis_index("c")
    lanes = lax.iota(jnp.int32, _LANES)

    sbufs = (s0, s1)
    obufs = (o0, o1)
    sem_s = (sem_s0, sem_s1)
    sem_w = (sem_w0, sem_w1)

    def _params(it):
        bi = wid + it * _NW
        valid = bi < _NBLK
        rc = pl.multiple_of(jnp.where(valid, bi, 0) * _RC, 128)
        gd = pl.multiple_of(jnp.where(valid, bi * _LPB, _NLINE), 8)
        return rc, gd

    def _fbody(f, carry):
        def _fire_stage(it):
            rc, _ = _params(it)
            return pltpu.async_copy(
                tab_hbm.at[f, :, pl.ds(rc, _RC)],
                sbufs[it % 2].at[:, pl.ds(0, _RC)],
                sem_s[it % 2]
            )

        seven = jnp.full((_LANES,), 7, jnp.int32)

        def _extract(sb, ob):
            def _cchunk(ci, carry2, sb=sb, ob=ob):
                c16 = lanes + ci * _LANES
                rows16 = lax.shift_right_logical(c16, 3)
                colb = lax.shift_left(lax.bitwise_and(c16, seven), 4)
                for d in range(_DIM):
                    vals = sb[d, pl.ds(ci * _LANES, _LANES)]
                    plsc.store_scatter(ob, [rows16, colb + d], vals)
                return carry2

            lax.fori_loop(0, _RC // _LANES, _cchunk, 0)

        def _fire_write(it):
            _, gd = _params(it)
            return pltpu.async_copy(
                obufs[it % 2].at[:, pl.ds(0, 128)],
                pk_hbm.at[f, pl.ds(gd, _LPB), :],
                sem_w[it % 2]
            )

        cp0 = _fire_stage(0)
        cp1 = _fire_stage(1)
        cp0.wait()
        _extract(sbufs[0], obufs[0])
        w0 = _fire_write(0)
        cp2 = _fire_stage(2)
        cp1.wait()
        _extract(sbufs[1], obufs[1])
        w1 = _fire_write(1)
        cp2.wait()
        w0.wait()
        _extract(sbufs[0], obufs[0])
        w2 = _fire_write(2)
        w1.wait()
        w2.wait()
        return carry

    lax.fori_loop(0, _NF, _fbody, 0)


def _lookup_body(idx_hbm, pk_hbm, tail_hbm, out_hbm,
                 idx_v, g_v, gs_v, s_v, rows_v, acc_v, sem):
    wid = lax.axis_index("s") * _NC + lax.axis_index("c")
    base = wid * _BPW

    seven = jnp.full((_LANES,), 7, jnp.int32)
    gmax = jnp.full((_LANES,), _NLINE - 1, jnp.int32)
    nchunk = _BPW // _LANES
    hchunk = _HB // _LANES
    inv = jnp.full((_LANES,), 1.0 / _NF, jnp.float32)
    lanes = lax.iota(jnp.int32, _LANES)

    zero = jnp.full((_LANES,), 0.0, jnp.float32)

    def _zero(j, carry):
        for d in range(_DIM):
            acc_v[d, pl.ds(j * _LANES, _LANES)] = zero
        return carry

    lax.fori_loop(0, nchunk, _zero, 0)

    def _feature(f, carry):
        pltpu.sync_copy(idx_hbm.at[f, pl.ds(base, _BPW)], idx_v)
        pltpu.sync_copy(tail_hbm.at[f], rows_v.at[pl.ds(_HB, _TROWS)])

        def _split(j, carry2):
            s = pl.ds(j * _LANES, _LANES)
            v = idx_v[s]
            g = lax.shift_right_logical(v, 3)
            g_v[s] = g
            gs_v[s] = jnp.minimum(g, gmax)
            s_v[s] = lax.shift_left(lax.bitwise_and(v, seven), 4)
            return carry2

        lax.fori_loop(0, nchunk, _split, 0)

        for h in range(2):
            copies = [
                pltpu.async_copy(
                    pk_hbm.at[f].at[gs_v.at[pl.ds(h * _HB + c * _GCH, _GCH)]],
                    rows_v.at[pl.ds(c * _GCH, _GCH)],
                    sem,
                )
                for c in range(_HB // _GCH)
            ]
            for cp in copies:
                cp.wait()

            def _ext(j, carry2, h=h):
                sl = pl.ds(h * _HB + j * _LANES, _LANES)
                g16 = g_v[sl]
                rows16 = jnp.where(
                    g16 > gmax, g16 + (_HB - _NLINE), lanes + j * _LANES
                )
                cols = s_v[sl]
                for d in range(_DIM):
                    acc_v[d, sl] = acc_v[d, sl] + plsc.load_gather(
                        rows_v, [rows16, cols + d]
                    )
                return carry2

            lax.fori_loop(0, hchunk, _ext, 0)
        return carry

    lax.fori_loop(0, _NF, _feature, 0)

    def _scale(j, carry):
        for d in range(_DIM):
            sl = pl.ds(j * _LANES, _LANES)
            acc_v[d, sl] = acc_v[d, sl] * inv
        return carry

    lax.fori_loop(0, nchunk, _scale, 0)
    pltpu.sync_copy(acc_v, out_hbm.at[:, pl.ds(base, _BPW)])


def kernel(inputs, batch_size, tables):
    del batch_size
    idx_t = inputs.T
    tab_t = jnp.transpose(tables, (0, 2, 1))
    tab_tail = jnp.pad(
        lax.slice(tables, (0, _TAIL0, 0), (_NF, _VOC, _DIM)),
        ((0, 0), (0, _TROWS * 8 - (_VOC - _TAIL0)), (0, 0)),
    ).reshape(_NF, _TROWS, 128)

    repack = pl.kernel(
        _repack_body,
        out_type=jax.ShapeDtypeStruct((_NF, _NLINE + _LPB, 128), jnp.float32),
        mesh=plsc.VectorSubcoreMesh(core_axis_name="c", subcore_axis_name="s"),
        scratch_types=[
            pltpu.VMEM((_DIM, _RC + 1), jnp.float32),
            pltpu.VMEM((_DIM, _RC + 1), jnp.float32),
            pltpu.VMEM((_LPB, 129), jnp.float32),
            pltpu.VMEM((_LPB, 129), jnp.float32),
            pltpu.SemaphoreType.DMA,
            pltpu.SemaphoreType.DMA,
            pltpu.SemaphoreType.DMA,
            pltpu.SemaphoreType.DMA,
        ],
        compiler_params=pltpu.CompilerParams(
            use_tc_tiling_on_sc=True, needs_layout_passes=False
        ),
    )
    packed = repack(tab_t)

    lookup = pl.kernel(
        _lookup_body,
        out_type=jax.ShapeDtypeStruct((_DIM, _B), jnp.float32),
        mesh=plsc.VectorSubcoreMesh(core_axis_name="c", subcore_axis_name="s"),
        scratch_types=[
            pltpu.VMEM((_BPW,), jnp.int32),
            pltpu.VMEM((_BPW,), jnp.int32),
            pltpu.VMEM((_BPW,), jnp.int32),
            pltpu.VMEM((_BPW,), jnp.int32),
            pltpu.VMEM((_HB + _TROWS, 128), jnp.float32),
            pltpu.VMEM((_DIM, _BPW), jnp.float32),
            pltpu.SemaphoreType.DMA,
        ],
        compiler_params=pltpu.CompilerParams(
            use_tc_tiling_on_sc=True, needs_layout_passes=False
        ),
    )
    out_t = lookup(idx_t, packed, tab_tail)
    return out_t.T
